# Initial kernel scaffold; baseline (speedup 1.0000x reference)
import jax
import jax.numpy as jnp
from jax.experimental import pallas as pl


def kernel(x, params, edge_index, batch):
    raise NotImplementedError("write your pallas kernel here")


# speedup vs baseline: 4.8876x; 4.8876x over previous
import functools

import jax
import jax.numpy as jnp
from jax import lax
from jax.experimental import pallas as pl
from jax.experimental.pallas import tpu as pltpu
from jax.experimental.pallas import tpu_sc as plsc

N_NODES = 100000
N_EDGES = 3200000
F_IN = 14
D = 32
HD = 16
G_SEGS = 1024
BN_EPS = 1e-5

NC = 2
NS = 16

CH = 1024
CHUNKS = 196
E_PAD = NS * CHUNKS * CH
E_ROWS = E_PAD // 128
ROWS_PER_SUB = E_ROWS // NS

HALF = N_NODES // NC
ACC_ROWS = 50048
ZERO_SPAN = ACC_ROWS // NS
OUT_SPAN = 3128
OUT_SPAN_LAST = HALF - (NS - 1) * OUT_SPAN


def _sc_agg(h_hbm, src2d, dstloc):
  mesh = plsc.VectorSubcoreMesh(core_axis_name="c", subcore_axis_name="s")

  @functools.partial(
      pl.kernel,
      out_type=jax.ShapeDtypeStruct((N_NODES, HD), jnp.float32),
      mesh=mesh,
      compiler_params=pltpu.CompilerParams(use_tc_tiling_on_sc=False),
      scratch_types=[
          pltpu.VMEM((8, 128), jnp.int32),
          pltpu.VMEM((8, 128), jnp.int32),
          pltpu.VMEM((CH, HD), jnp.float32),
          pltpu.VMEM_SHARED((ACC_ROWS, HD), jnp.float32),
          pltpu.SemaphoreType.DMA,
      ],
  )
  def agg_kernel(h_ref, src_ref, dst_ref, agg_ref,
                 idx_s, idx_d, rows, accum, sem):
    c = lax.axis_index("c")
    s = lax.axis_index("s")
    base = c * HALF

    zvec = jnp.zeros((HD,), jnp.float32)

    @pl.loop(0, CH)
    def _(i):
      rows[i, pl.ds(0, HD)] = zvec

    zoff = s * ZERO_SPAN
    done = 0
    for span in (1024, 1024, 1024, ZERO_SPAN - 3072):
      pltpu.sync_copy(rows.at[pl.ds(0, span)],
                      accum.at[pl.ds(zoff + done, span)])
      done += span
    plsc.subcore_barrier()

    row_base = s * ROWS_PER_SUB

    @pl.loop(0, CHUNKS)
    def _(chunk):
      row0 = row_base + chunk * 8
      pltpu.sync_copy(src_ref.at[pl.ds(row0, 8)], idx_s)
      pltpu.sync_copy(dst_ref.at[c].at[pl.ds(row0, 8)], idx_d)
      cps = [
          pltpu.async_copy(h_ref.at[idx_s.at[j]],
                           rows.at[pl.ds(j * 128, 128)], sem)
          for j in range(8)
      ]
      for cp in cps:
        cp.wait()
      for j in range(8):
        pltpu.sync_copy(rows.at[pl.ds(j * 128, 128)],
                        accum.at[idx_d.at[j]], add=True)

    plsc.subcore_barrier()
    ooff = s * OUT_SPAN

    @pl.when(s < NS - 1)
    def _():
      pltpu.sync_copy(accum.at[pl.ds(ooff, OUT_SPAN)],
                      agg_ref.at[pl.ds(base + ooff, OUT_SPAN)])

    @pl.when(s == NS - 1)
    def _():
      pltpu.sync_copy(accum.at[pl.ds(ooff, OUT_SPAN_LAST)],
                      agg_ref.at[pl.ds(base + ooff, OUT_SPAN_LAST)])

  return agg_kernel(h_hbm, src2d, dstloc)


_HIGH = jax.lax.Precision.HIGHEST


def _dot(a, b):
  return jnp.dot(a, b, preferred_element_type=jnp.float32, precision=_HIGH)


def _remap_body(d_ref, o_ref):
  c = pl.program_id(0)
  dv = d_ref[0, :, :]
  loc = dv - c * HALF
  ok = (loc >= 0) & (loc < HALF)
  o_ref[0, :, :] = jnp.where(ok, loc, HALF)


_remap = pl.pallas_call(
    _remap_body,
    grid=(NC, E_ROWS // 256),
    in_specs=[pl.BlockSpec((1, 256, 128), lambda c, i: (0, i, 0))],
    out_specs=pl.BlockSpec((1, 256, 128), lambda c, i: (c, i, 0)),
    out_shape=jax.ShapeDtypeStruct((NC, E_ROWS, 128), jnp.int32),
)


def _mlp_body(split_in, h_refs, w1_ref, b1_ref, w2_ref, b2_ref,
              lo_ref, hi_ref, full_ref):
  if split_in:
    hlo, hhi, alo, ahi = h_refs
    hb = jnp.concatenate([hlo[...] + alo[...], hhi[...] + ahi[...]], axis=1)
  else:
    h, a = h_refs
    hb = h[...] + a[...]
  h1 = jnp.maximum(_dot(hb, w1_ref[...]) + b1_ref[...], 0.0)
  h2 = jnp.maximum(_dot(h1, w2_ref[...]) + b2_ref[...], 0.0)
  lo_ref[...] = h2[:, :HD]
  hi_ref[...] = h2[:, HD:]
  full_ref[...] = h2


def _make_mlp(split_in):
  br = 2000
  grid = N_NODES // br
  full = lambda i: (0, 0)
  n_in = 4 if split_in else 2
  fin = HD if split_in else HD

  def body(*refs):
    h_refs = refs[:n_in]
    w1_ref, b1_ref, w2_ref, b2_ref, lo_ref, hi_ref, full_ref = refs[n_in:]
    _mlp_body(split_in, h_refs, w1_ref, b1_ref, w2_ref, b2_ref,
              lo_ref, hi_ref, full_ref)

  w1_rows = D if split_in else HD
  return pl.pallas_call(
      body,
      grid=(grid,),
      in_specs=[pl.BlockSpec((br, fin), lambda i: (i, 0))] * n_in + [
          pl.BlockSpec((w1_rows, D), full),
          pl.BlockSpec((1, D), full),
          pl.BlockSpec((D, D), full),
          pl.BlockSpec((1, D), full),
      ],
      out_specs=[
          pl.BlockSpec((br, HD), lambda i: (i, 0)),
          pl.BlockSpec((br, HD), lambda i: (i, 0)),
          pl.BlockSpec((br, D), lambda i: (i, 0)),
      ],
      out_shape=[
          jax.ShapeDtypeStruct((N_NODES, HD), jnp.float32),
          jax.ShapeDtypeStruct((N_NODES, HD), jnp.float32),
          jax.ShapeDtypeStruct((N_NODES, D), jnp.float32),
      ],
  )


_mlp_first = _make_mlp(False)
_mlp_rest = _make_mlp(True)


def _pool_head_body(h_ref, b_ref, w1_ref, b1_ref, w2_ref, b2_ref,
                    pool_ref, z_ref):
  i = pl.program_id(0)
  br = h_ref.shape[0]
  grid = pl.num_programs(0)

  @pl.when(i == 0)
  def _():
    pool_ref[...] = jnp.zeros_like(pool_ref)

  seg = b_ref[0, 0, :]
  ids = lax.broadcasted_iota(jnp.int32, (G_SEGS, br), 0)
  onehot = (ids == seg[None, :]).astype(jnp.float32)
  pool_ref[...] += _dot(onehot, h_ref[...])

  @pl.when(i == grid - 1)
  def _():
    z = jnp.maximum(_dot(pool_ref[...], w1_ref[...]) + b1_ref[...], 0.0)
    z_ref[...] = _dot(z, w2_ref[...]) + b2_ref[...]


_POOL_BR = 2000
_pool_head = pl.pallas_call(
    _pool_head_body,
    grid=(N_NODES // _POOL_BR,),
    in_specs=[
        pl.BlockSpec((_POOL_BR, D), lambda i: (i, 0)),
        pl.BlockSpec((1, 1, _POOL_BR), lambda i: (i, 0, 0)),
        pl.BlockSpec((D, D), lambda i: (0, 0)),
        pl.BlockSpec((1, D), lambda i: (0, 0)),
        pl.BlockSpec((D, 1), lambda i: (0, 0)),
        pl.BlockSpec((1, 1), lambda i: (0, 0)),
    ],
    out_specs=[
        pl.BlockSpec((G_SEGS, D), lambda i: (0, 0)),
        pl.BlockSpec((G_SEGS, 1), lambda i: (0, 0)),
    ],
    out_shape=[
        jax.ShapeDtypeStruct((G_SEGS, D), jnp.float32),
        jax.ShapeDtypeStruct((G_SEGS, 1), jnp.float32),
    ],
)


def kernel(x, params, edge_index, batch):
  src, dst = edge_index[0], edge_index[1]
  pad = E_PAD - N_EDGES
  src2d = jnp.concatenate([src, jnp.zeros((pad,), jnp.int32)]).reshape(-1, 128)
  dst3d = jnp.concatenate(
      [dst, jnp.full((pad,), N_NODES, jnp.int32)]).reshape(1, -1, 128)
  dstloc = _remap(dst3d)

  inv = 1.0 / jnp.sqrt(1.0 + BN_EPS)
  h_lo = jnp.pad(x, ((0, 0), (0, HD - F_IN)))
  h_hi = None
  h_full = None
  for i in range(5):
    p = params[f"conv{i + 1}"]
    scale = p["gamma"] * inv
    w1 = p["W1"] * scale[None, :]
    b1 = (p["b1"] * scale + p["beta"])[None, :]
    w2 = p["W2"]
    b2 = p["b2"][None, :]
    if i == 0:
      w1 = jnp.pad(w1, ((0, HD - F_IN), (0, 0)))
      agg = _sc_agg(h_lo, src2d, dstloc)
      h_lo, h_hi, h_full = _mlp_first(h_lo, agg, w1, b1, w2, b2)
    else:
      agg_lo = _sc_agg(h_lo, src2d, dstloc)
      agg_hi = _sc_agg(h_hi, src2d, dstloc)
      h_lo, h_hi, h_full = _mlp_rest(h_lo, h_hi, agg_lo, agg_hi,
                                     w1, b1, w2, b2)

  batch3d = batch.reshape(N_NODES // _POOL_BR, 1, _POOL_BR)
  _, z = _pool_head(h_full, batch3d, params["lin1_W"],
                    params["lin1_b"][None, :], params["lin2_W"],
                    params["lin2_b"][None, :])
  return (z.reshape(-1), h_full)
